# baseline (speedup 1.0000x reference)
import functools
import jax
import jax.numpy as jnp
from jax import lax
from jax.experimental import pallas as pl
from jax.experimental.pallas import tpu as pltpu
from jax.experimental.pallas import tpu_sc as plsc

N = 10000
E = 320000
F = 128
C_OUT = 10
G = 16
SIGMA_SQ = 0.25
KBN = 1.0 / (1.0 + 1e-5) ** 0.5

NC = 2
NS = 16
NW = NC * NS
EW_PER = E // NW
K = 128
CH = 80
PER = CH * K
FULL = EW_PER // K
TAIL = EW_PER - FULL * K

DEG_STRIPE = 640
N_PAD = NS * DEG_STRIPE
ROW_STRIPE = 624
ROW_LAST = N - 15 * ROW_STRIPE
HF = F // 2
CH2 = 2 * CH
T4 = CH2 // 4

_sc_mesh = plsc.VectorSubcoreMesh(core_axis_name="c", subcore_axis_name="s")


def _edge_body(uv3, rawx, ew3, deg2, uv2d, v2d, ew2d, rawx_v, zbuf, deg_acc,
               sem):
    cid = lax.axis_index("c")
    sid = lax.axis_index("s")
    wid = sid * NC + cid

    pltpu.sync_copy(uv3.at[wid], uv2d)
    pltpu.sync_copy(rawx, rawx_v)

    zero = jnp.zeros((16,), jnp.float32)
    for i in range(40):
        zbuf[pl.ds(i * 16, 16)] = zero
    pltpu.sync_copy(zbuf.at[pl.ds(0, DEG_STRIPE)],
                    deg_acc.at[pl.ds(sid * DEG_STRIPE, DEG_STRIPE)])
    plsc.subcore_barrier()

    def chunk(j, _):
        for c in range(K // 16):
            uvv = uv2d[j, pl.ds(c * 16, 16)]
            uvec = uvv & 0xFFFF
            vvec = lax.shift_right_logical(uvv, 16)
            v2d[j, pl.ds(c * 16, 16)] = vvec
            ub = uvec * 4
            vb = vvec * 4
            d2 = jnp.zeros((16,), jnp.float32)
            for k in range(4):
                au = plsc.load_gather(rawx_v, [ub + k])
                av = plsc.load_gather(rawx_v, [vb + k])
                d = au - av
                d2 = d2 + d * d
            ew2d[j, pl.ds(c * 16, 16)] = jnp.exp(d2 * (-0.5 / SIGMA_SQ))
        return 0

    lax.fori_loop(0, CH, chunk, 0)

    for t in range(TAIL // 16, K // 16):
        ew2d[FULL, pl.ds(t * 16, 16)] = zero
    for j in range(FULL + 1, CH):
        for t in range(K // 16):
            ew2d[j, pl.ds(t * 16, 16)] = zero

    def dbatch(t, _):
        for b in range(16):
            pltpu.async_copy(ew2d.at[16 * t + b],
                             deg_acc.at[v2d.at[16 * t + b]], sem, add=True)
        for b in range(16):
            pltpu.make_async_copy(ew2d.at[16 * t + b],
                                  deg_acc.at[v2d.at[16 * t + b]], sem).wait()
        return 0

    lax.fori_loop(0, CH // 16, dbatch, 0)

    pltpu.sync_copy(ew2d, ew3.at[wid])
    plsc.subcore_barrier()

    pltpu.sync_copy(deg_acc.at[pl.ds(sid * DEG_STRIPE, DEG_STRIPE)],
                    deg2.at[cid, 0, pl.ds(sid * DEG_STRIPE, DEG_STRIPE)])


_edge_kernel = functools.partial(
    pl.kernel,
    _edge_body,
    out_type=[jax.ShapeDtypeStruct((NW, CH, K), jnp.float32),
              jax.ShapeDtypeStruct((NC, 1, N_PAD), jnp.float32)],
    mesh=_sc_mesh,
    scratch_types=[
        pltpu.VMEM((CH, K), jnp.int32),
        pltpu.VMEM((CH, K), jnp.int32),
        pltpu.VMEM((CH, K), jnp.float32),
        pltpu.VMEM((N * 4,), jnp.float32),
        pltpu.VMEM((DEG_STRIPE,), jnp.float32),
        pltpu.VMEM_SHARED((N_PAD,), jnp.float32),
        pltpu.SemaphoreType.DMA,
    ],
    compiler_params=pltpu.CompilerParams(needs_layout_passes=False),
)()


def _conv_body(y2, uv3, ew3, deg2, out2, uv2d, ew2d, uring, vring, ering,
               dinv_t, rows0, rows1, rows2, rows3, acc, gsem0, gsem1, gsem2,
               gsem3, ssem0, ssem1, ssem2, ssem3):
    cid = lax.axis_index("c")
    sid = lax.axis_index("s")

    pltpu.sync_copy(deg2.at[0], dinv_t)
    pltpu.sync_copy(deg2.at[1], ew2d.at[pl.ds(0, CH)])

    def rsq(i, _):
        r = i // (K // 16)
        c = i % (K // 16)
        d = (dinv_t[r, pl.ds(c * 16, 16)] + ew2d[r, pl.ds(c * 16, 16)]
             + 1.0)
        ih = 0x5F3759DF - lax.shift_right_logical(
            plsc.bitcast(d, jnp.int32), 1)
        xx = plsc.bitcast(ih, jnp.float32)
        for _it in range(3):
            xx = xx * (1.5 - 0.5 * d * xx * xx)
        dinv_t[r, pl.ds(c * 16, 16)] = xx
        return 0

    lax.fori_loop(0, (N_PAD // K) * (K // 16), rsq, 0)

    pltpu.sync_copy(uv3.at[2 * sid], uv2d.at[pl.ds(0, CH)])
    pltpu.sync_copy(uv3.at[2 * sid + 1], uv2d.at[pl.ds(CH, CH)])
    pltpu.sync_copy(ew3.at[2 * sid], ew2d.at[pl.ds(0, CH)])
    pltpu.sync_copy(ew3.at[2 * sid + 1], ew2d.at[pl.ds(CH, CH)])

    off = cid * N

    def prep(j, slot):
        for g in range(K // 16):
            uvv = uv2d[j, pl.ds(g * 16, 16)]
            uu = uvv & 0xFFFF
            uring[slot, pl.ds(g * 16, 16)] = uu + off
            vring[slot, pl.ds(g * 16, 16)] = lax.shift_right_logical(uvv, 16)
            dv = plsc.load_gather(
                dinv_t, [lax.shift_right_logical(uu, 7), uu & 0x7F])
            ering[slot, pl.ds(g * 16, 16)] = ew2d[j, pl.ds(g * 16, 16)] * dv

    zero = jnp.zeros((16,), jnp.float32)

    def zrow(r, _):
        for c in range(HF // 16):
            rows0[r, pl.ds(c * 16, 16)] = zero
        return 0

    lax.fori_loop(0, K, zrow, 0)

    base = sid * ROW_STRIPE

    @pl.when(sid < 15)
    def _():
        for t in range(4):
            pltpu.sync_copy(rows0, acc.at[pl.ds(base + t * K, K)])
        pltpu.sync_copy(rows0.at[pl.ds(0, ROW_STRIPE - 4 * K)],
                        acc.at[pl.ds(base + 4 * K, ROW_STRIPE - 4 * K)])

    @pl.when(sid == 15)
    def _():
        for t in range(5):
            pltpu.sync_copy(rows0, acc.at[pl.ds(base + t * K, K)])
    plsc.subcore_barrier()

    bufs = ((rows0, gsem0, ssem0), (rows1, gsem1, ssem1),
            (rows2, gsem2, ssem2), (rows3, gsem3, ssem3))

    def scale(rows, slot):
        @plsc.parallel_loop(0, K // 16, unroll=2)
        def sgroup(g):
            ewv = ering[slot, pl.ds(g * 16, 16)]
            for r16 in range(16):
                sc = ewv[r16]
                r = g * 16 + r16
                for c in range(HF // 16):
                    rows[r, pl.ds(c * 16, 16)] = (
                        rows[r, pl.ds(c * 16, 16)] * sc)

    prep(0, 0)
    prep(1, 1)
    pltpu.async_copy(y2.at[uring.at[0]], rows0, gsem0)
    pltpu.async_copy(y2.at[uring.at[1]], rows1, gsem1)

    def quad(t, _):
        for b in range(4):
            rows, gsem, ssem = bufs[b]
            ns = (b + 2) % 4
            nrows, ngsem, nssem = bufs[ns]
            j = 4 * t + b
            pltpu.make_async_copy(y2.at[uring.at[b]], rows, gsem).wait()
            if b < 2:
                @pl.when(t > 0)
                def _():
                    pltpu.make_async_copy(
                        nrows, acc.at[vring.at[ns]], nssem).wait()
                prep(j + 2, ns)
                pltpu.async_copy(y2.at[uring.at[ns]], nrows, ngsem)
            else:
                @pl.when(t < T4 - 1)
                def _():
                    pltpu.make_async_copy(
                        nrows, acc.at[vring.at[ns]], nssem).wait()
                    prep(j + 2, ns)
                    pltpu.async_copy(y2.at[uring.at[ns]], nrows, ngsem)
            scale(rows, b)
            pltpu.async_copy(rows, acc.at[vring.at[b]], ssem, add=True)
        return 0

    lax.fori_loop(0, T4, quad, 0)
    for b in range(4):
        rows, gsem, ssem = bufs[b]
        pltpu.make_async_copy(rows, acc.at[vring.at[b]], ssem).wait()
    plsc.subcore_barrier()

    @pl.when(sid < 15)
    def _():
        pltpu.sync_copy(acc.at[pl.ds(base, ROW_STRIPE)],
                        out2.at[cid, pl.ds(base, ROW_STRIPE)])

    @pl.when(sid == 15)
    def _():
        pltpu.sync_copy(acc.at[pl.ds(base, ROW_LAST)],
                        out2.at[cid, pl.ds(base, ROW_LAST)])


_conv_kernel = functools.partial(
    pl.kernel,
    _conv_body,
    out_type=jax.ShapeDtypeStruct((NC, N, HF), jnp.float32),
    mesh=_sc_mesh,
    scratch_types=[
        pltpu.VMEM((CH2, K), jnp.int32),
        pltpu.VMEM((CH2, K), jnp.float32),
        pltpu.VMEM((4, K), jnp.int32),
        pltpu.VMEM((4, K), jnp.int32),
        pltpu.VMEM((4, K), jnp.float32),
        pltpu.VMEM((N_PAD // K, K), jnp.float32),
        pltpu.VMEM((K, HF), jnp.float32),
        pltpu.VMEM((K, HF), jnp.float32),
        pltpu.VMEM((K, HF), jnp.float32),
        pltpu.VMEM((K, HF), jnp.float32),
        pltpu.VMEM_SHARED((N, HF), jnp.float32),
        pltpu.SemaphoreType.DMA,
        pltpu.SemaphoreType.DMA,
        pltpu.SemaphoreType.DMA,
        pltpu.SemaphoreType.DMA,
        pltpu.SemaphoreType.DMA,
        pltpu.SemaphoreType.DMA,
        pltpu.SemaphoreType.DMA,
        pltpu.SemaphoreType.DMA,
    ],
    compiler_params=pltpu.CompilerParams(needs_layout_passes=False,
                                         use_tc_tiling_on_sc=False),
)()


RB = 1000
NB = N // RB


def _mm_body(x_ref, w_ref, o_ref, h_ref):
    xw = jnp.dot(x_ref[...], w_ref[...], preferred_element_type=jnp.float32)
    o_ref[...] = xw
    h_ref[0] = xw[:, :HF]
    h_ref[1] = xw[:, HF:]


_mm = pl.pallas_call(
    _mm_body,
    grid=(NB,),
    in_specs=[pl.BlockSpec((RB, F), lambda i: (i, 0)),
              pl.BlockSpec((F, F), lambda i: (0, 0))],
    out_specs=[pl.BlockSpec((RB, F), lambda i: (i, 0)),
               pl.BlockSpec((NC, RB, HF), lambda i: (0, i, 0))],
    out_shape=[jax.ShapeDtypeStruct((N, F), jnp.float32),
               jax.ShapeDtypeStruct((NC, N, HF), jnp.float32)],
)


def _mid_body(acc_ref, xw_ref, d0_ref, d1_ref, b1_ref, g1_ref, be1_ref,
              w2_ref, xw2_ref, y2_ref):
    dinv = lax.rsqrt(d0_ref[0, 0, :] + d1_ref[0, 0, :] + 1.0)
    aggs = jnp.concatenate([acc_ref[0], acc_ref[1]], axis=1)
    agg = (aggs * dinv[:, None]
           + xw_ref[...] * (dinv * dinv)[:, None] + b1_ref[...])
    h = jnp.maximum(agg * KBN * g1_ref[...] + be1_ref[...], 0.0)
    xw2 = jnp.dot(h, w2_ref[...], preferred_element_type=jnp.float32)
    xw2_ref[...] = xw2
    y2_ref[0] = xw2[:, :HF]
    y2_ref[1] = xw2[:, HF:]


_mid = pl.pallas_call(
    _mid_body,
    grid=(NB,),
    in_specs=[pl.BlockSpec((NC, RB, HF), lambda i: (0, i, 0)),
              pl.BlockSpec((RB, F), lambda i: (i, 0)),
              pl.BlockSpec((1, 1, RB), lambda i: (i, 0, 0)),
              pl.BlockSpec((1, 1, RB), lambda i: (i, 0, 0)),
              pl.BlockSpec((1, F), lambda i: (0, 0)),
              pl.BlockSpec((1, F), lambda i: (0, 0)),
              pl.BlockSpec((1, F), lambda i: (0, 0)),
              pl.BlockSpec((F, F), lambda i: (0, 0))],
    out_specs=[pl.BlockSpec((RB, F), lambda i: (i, 0)),
               pl.BlockSpec((NC, RB, HF), lambda i: (0, i, 0))],
    out_shape=[jax.ShapeDtypeStruct((N, F), jnp.float32),
               jax.ShapeDtypeStruct((NC, N, HF), jnp.float32)],
)


def _head_body(acc_ref, xw_ref, d0_ref, d1_ref, b2_ref, g2_ref, be2_ref,
               batch_ref, wc1_ref, bc1_ref, wc2_ref, bc2_ref,
               out_ref, pooled_ref, cnt_ref):
    i = pl.program_id(0)

    @pl.when(i == 0)
    def _():
        pooled_ref[...] = jnp.zeros((G, F), jnp.float32)
        cnt_ref[...] = jnp.zeros((G, F), jnp.float32)

    dinv = lax.rsqrt(d0_ref[0, 0, :] + d1_ref[0, 0, :] + 1.0)
    aggs = jnp.concatenate([acc_ref[0], acc_ref[1]], axis=1)
    agg = (aggs * dinv[:, None]
           + xw_ref[...] * (dinv * dinv)[:, None] + b2_ref[...])
    h = jnp.maximum(agg * KBN * g2_ref[...] + be2_ref[...], 0.0)
    b = batch_ref[0, 0, :]
    oneh = (b[:, None] == lax.broadcasted_iota(jnp.int32, (RB, G), 1)
            ).astype(jnp.float32)
    dn = (((0,), (0,)), ((), ()))
    pooled_ref[...] += lax.dot_general(oneh, h, dn,
                                       preferred_element_type=jnp.float32)
    cnt_ref[...] += lax.dot_general(oneh, jnp.ones((RB, F), jnp.float32), dn,
                                    preferred_element_type=jnp.float32)

    @pl.when(i == NB - 1)
    def _():
        pooled = pooled_ref[...] / jnp.maximum(cnt_ref[...], 1.0)
        z = jnp.maximum(
            jnp.dot(pooled, wc1_ref[...], preferred_element_type=jnp.float32)
            + bc1_ref[...], 0.0)
        out_ref[...] = jnp.dot(z, wc2_ref[...],
                               preferred_element_type=jnp.float32) + bc2_ref[...]


_head = pl.pallas_call(
    _head_body,
    grid=(NB,),
    in_specs=[pl.BlockSpec((NC, RB, HF), lambda i: (0, i, 0)),
              pl.BlockSpec((RB, F), lambda i: (i, 0)),
              pl.BlockSpec((1, 1, RB), lambda i: (i, 0, 0)),
              pl.BlockSpec((1, 1, RB), lambda i: (i, 0, 0)),
              pl.BlockSpec((1, F), lambda i: (0, 0)),
              pl.BlockSpec((1, F), lambda i: (0, 0)),
              pl.BlockSpec((1, F), lambda i: (0, 0)),
              pl.BlockSpec((1, 1, RB), lambda i: (i, 0, 0)),
              pl.BlockSpec((F, F), lambda i: (0, 0)),
              pl.BlockSpec((1, F), lambda i: (0, 0)),
              pl.BlockSpec((F, F), lambda i: (0, 0)),
              pl.BlockSpec((1, F), lambda i: (0, 0))],
    out_specs=pl.BlockSpec((G, F), lambda i: (0, 0)),
    out_shape=jax.ShapeDtypeStruct((G, F), jnp.float32),
    scratch_shapes=[pltpu.VMEM((G, F), jnp.float32),
                    pltpu.VMEM((G, F), jnp.float32)],
)


def kernel(x, edge_index, raw_x, batch, W1, b1, g1, be1, W2, b2, g2, be2,
           Wc1, bc1, Wc2, bc2):
    f32 = jnp.float32
    u = edge_index[0].astype(jnp.int32).reshape(NW, EW_PER)
    v = edge_index[1].astype(jnp.int32).reshape(NW, EW_PER)
    pad = (jnp.arange(PER - EW_PER, dtype=jnp.int32)[None, :] * 89
           + jnp.arange(NW, dtype=jnp.int32)[:, None] * 313) % N
    u3 = jnp.concatenate([u, pad], axis=1)
    v3 = jnp.concatenate([v, pad], axis=1)
    uv3 = (u3 | (v3 << 16)).reshape(NW, CH, K)
    rawx_flat = raw_x.astype(f32).reshape(N * 4)

    xw1, xw1h = _mm(x.astype(f32), W1)
    ew3, deg2 = _edge_kernel(uv3, rawx_flat)

    deg2r = deg2.reshape(NC, N_PAD // K, K)
    acc1 = _conv_kernel(xw1h.reshape(NC * N, HF), uv3, ew3, deg2r)

    d03 = deg2[0, 0, :N].reshape(NB, 1, RB)
    d13 = deg2[1, 0, :N].reshape(NB, 1, RB)
    r1 = b1.reshape(1, F)
    xw2, y2h = _mid(acc1, xw1, d03, d13, r1, g1.reshape(1, F),
                    be1.reshape(1, F), W2)

    acc2 = _conv_kernel(y2h.reshape(NC * N, HF), uv3, ew3, deg2r)

    batch3 = batch.astype(jnp.int32).reshape(NB, 1, RB)
    wc2p = jnp.zeros((F, F), f32).at[:, :C_OUT].set(Wc2)
    bc2p = jnp.zeros((1, F), f32).at[0, :C_OUT].set(bc2)
    out = _head(acc2, xw2, d03, d13, b2.reshape(1, F), g2.reshape(1, F),
                be2.reshape(1, F), batch3, Wc1, bc1.reshape(1, F), wc2p, bc2p)
    return out[:, :C_OUT]


# speedup vs baseline: 1.0527x; 1.0527x over previous
import functools
import jax
import jax.numpy as jnp
from jax import lax
from jax.experimental import pallas as pl
from jax.experimental.pallas import tpu as pltpu
from jax.experimental.pallas import tpu_sc as plsc

N = 10000
E = 320000
F = 128
C_OUT = 10
G = 16
SIGMA_SQ = 0.25
KBN = 1.0 / (1.0 + 1e-5) ** 0.5

NC = 2
NS = 16
NW = NC * NS
EW_PER = E // NW
K = 128
CH = 80
PER = CH * K
FULL = EW_PER // K
TAIL = EW_PER - FULL * K

DEG_STRIPE = 640
N_PAD = NS * DEG_STRIPE
ROW_STRIPE = 624
ROW_LAST = N - 15 * ROW_STRIPE
HF = F // 2
CH2 = 2 * CH
T4 = CH2 // 4

_sc_mesh = plsc.VectorSubcoreMesh(core_axis_name="c", subcore_axis_name="s")


def _edge_body(uv3, rawx, ew3, deg2, uv2d, v2d, ew2d, rawx_v, zbuf, deg_acc,
               sem):
    cid = lax.axis_index("c")
    sid = lax.axis_index("s")
    wid = sid * NC + cid

    pltpu.sync_copy(uv3.at[wid], uv2d)
    pltpu.sync_copy(rawx, rawx_v)

    zero = jnp.zeros((16,), jnp.float32)
    for i in range(40):
        zbuf[pl.ds(i * 16, 16)] = zero
    pltpu.sync_copy(zbuf.at[pl.ds(0, DEG_STRIPE)],
                    deg_acc.at[pl.ds(sid * DEG_STRIPE, DEG_STRIPE)])
    plsc.subcore_barrier()

    def chunk(j, _):
        for c in range(K // 16):
            uvv = uv2d[j, pl.ds(c * 16, 16)]
            uvec = uvv & 0xFFFF
            vvec = lax.shift_right_logical(uvv, 16)
            v2d[j, pl.ds(c * 16, 16)] = vvec
            ub = uvec * 4
            vb = vvec * 4
            d2 = jnp.zeros((16,), jnp.float32)
            for k in range(4):
                au = plsc.load_gather(rawx_v, [ub + k])
                av = plsc.load_gather(rawx_v, [vb + k])
                d = au - av
                d2 = d2 + d * d
            ew2d[j, pl.ds(c * 16, 16)] = jnp.exp(d2 * (-0.5 / SIGMA_SQ))
        return 0

    lax.fori_loop(0, CH, chunk, 0)

    for t in range(TAIL // 16, K // 16):
        ew2d[FULL, pl.ds(t * 16, 16)] = zero
    for j in range(FULL + 1, CH):
        for t in range(K // 16):
            ew2d[j, pl.ds(t * 16, 16)] = zero

    def dbatch(t, _):
        for b in range(16):
            pltpu.async_copy(ew2d.at[16 * t + b],
                             deg_acc.at[v2d.at[16 * t + b]], sem, add=True)
        for b in range(16):
            pltpu.make_async_copy(ew2d.at[16 * t + b],
                                  deg_acc.at[v2d.at[16 * t + b]], sem).wait()
        return 0

    lax.fori_loop(0, CH // 16, dbatch, 0)

    pltpu.sync_copy(ew2d, ew3.at[wid])
    plsc.subcore_barrier()

    pltpu.sync_copy(deg_acc.at[pl.ds(sid * DEG_STRIPE, DEG_STRIPE)],
                    deg2.at[cid, 0, pl.ds(sid * DEG_STRIPE, DEG_STRIPE)])


_edge_kernel = functools.partial(
    pl.kernel,
    _edge_body,
    out_type=[jax.ShapeDtypeStruct((NW, CH, K), jnp.float32),
              jax.ShapeDtypeStruct((NC, 1, N_PAD), jnp.float32)],
    mesh=_sc_mesh,
    scratch_types=[
        pltpu.VMEM((CH, K), jnp.int32),
        pltpu.VMEM((CH, K), jnp.int32),
        pltpu.VMEM((CH, K), jnp.float32),
        pltpu.VMEM((N * 4,), jnp.float32),
        pltpu.VMEM((DEG_STRIPE,), jnp.float32),
        pltpu.VMEM_SHARED((N_PAD,), jnp.float32),
        pltpu.SemaphoreType.DMA,
    ],
    compiler_params=pltpu.CompilerParams(needs_layout_passes=False),
)()


def _conv_body(y2, uv3, ew3, out2, uv2d, ew2d, uring, vring, rows0, rows1,
               rows2, rows3, acc, gsem0, gsem1, gsem2, gsem3, ssem0, ssem1,
               ssem2, ssem3):
    cid = lax.axis_index("c")
    sid = lax.axis_index("s")

    pltpu.sync_copy(uv3.at[2 * sid], uv2d.at[pl.ds(0, CH)])
    pltpu.sync_copy(uv3.at[2 * sid + 1], uv2d.at[pl.ds(CH, CH)])
    pltpu.sync_copy(ew3.at[2 * sid], ew2d.at[pl.ds(0, CH)])
    pltpu.sync_copy(ew3.at[2 * sid + 1], ew2d.at[pl.ds(CH, CH)])

    off = cid * N

    def prep(j, slot):
        for g in range(K // 16):
            uvv = uv2d[j, pl.ds(g * 16, 16)]
            uring[slot, pl.ds(g * 16, 16)] = (uvv & 0xFFFF) + off
            vring[slot, pl.ds(g * 16, 16)] = lax.shift_right_logical(uvv, 16)

    zero = jnp.zeros((16,), jnp.float32)

    def zrow(r, _):
        for c in range(HF // 16):
            rows0[r, pl.ds(c * 16, 16)] = zero
        return 0

    lax.fori_loop(0, K, zrow, 0)

    base = sid * ROW_STRIPE

    @pl.when(sid < 15)
    def _():
        for t in range(4):
            pltpu.sync_copy(rows0, acc.at[pl.ds(base + t * K, K)])
        pltpu.sync_copy(rows0.at[pl.ds(0, ROW_STRIPE - 4 * K)],
                        acc.at[pl.ds(base + 4 * K, ROW_STRIPE - 4 * K)])

    @pl.when(sid == 15)
    def _():
        for t in range(5):
            pltpu.sync_copy(rows0, acc.at[pl.ds(base + t * K, K)])
    plsc.subcore_barrier()

    bufs = ((rows0, gsem0, ssem0), (rows1, gsem1, ssem1),
            (rows2, gsem2, ssem2), (rows3, gsem3, ssem3))

    def scale(rows, j):
        @plsc.parallel_loop(0, K // 16, unroll=4)
        def sgroup(g):
            ewv = ew2d[j, pl.ds(g * 16, 16)]
            for r16 in range(16):
                sc = ewv[r16]
                r = g * 16 + r16
                for c in range(HF // 16):
                    rows[r, pl.ds(c * 16, 16)] = (
                        rows[r, pl.ds(c * 16, 16)] * sc)

    prep(0, 0)
    prep(1, 1)
    pltpu.async_copy(y2.at[uring.at[0]], rows0, gsem0)
    pltpu.async_copy(y2.at[uring.at[1]], rows1, gsem1)

    def quad(t, _):
        for b in range(4):
            rows, gsem, ssem = bufs[b]
            ns = (b + 2) % 4
            nrows, ngsem, nssem = bufs[ns]
            j = 4 * t + b
            pltpu.make_async_copy(y2.at[uring.at[b]], rows, gsem).wait()
            if b < 2:
                @pl.when(t > 0)
                def _():
                    pltpu.make_async_copy(
                        nrows, acc.at[vring.at[ns]], nssem).wait()
                prep(j + 2, ns)
                pltpu.async_copy(y2.at[uring.at[ns]], nrows, ngsem)
            else:
                @pl.when(t < T4 - 1)
                def _():
                    pltpu.make_async_copy(
                        nrows, acc.at[vring.at[ns]], nssem).wait()
                    prep(j + 2, ns)
                    pltpu.async_copy(y2.at[uring.at[ns]], nrows, ngsem)
            scale(rows, j)
            pltpu.async_copy(rows, acc.at[vring.at[b]], ssem, add=True)
        return 0

    lax.fori_loop(0, T4, quad, 0)
    for b in range(4):
        rows, gsem, ssem = bufs[b]
        pltpu.make_async_copy(rows, acc.at[vring.at[b]], ssem).wait()
    plsc.subcore_barrier()

    @pl.when(sid < 15)
    def _():
        pltpu.sync_copy(acc.at[pl.ds(base, ROW_STRIPE)],
                        out2.at[cid, pl.ds(base, ROW_STRIPE)])

    @pl.when(sid == 15)
    def _():
        pltpu.sync_copy(acc.at[pl.ds(base, ROW_LAST)],
                        out2.at[cid, pl.ds(base, ROW_LAST)])


_conv_kernel = functools.partial(
    pl.kernel,
    _conv_body,
    out_type=jax.ShapeDtypeStruct((NC, N, HF), jnp.float32),
    mesh=_sc_mesh,
    scratch_types=[
        pltpu.VMEM((CH2, K), jnp.int32),
        pltpu.VMEM((CH2, K), jnp.float32),
        pltpu.VMEM((4, K), jnp.int32),
        pltpu.VMEM((4, K), jnp.int32),
        pltpu.VMEM((K, HF), jnp.float32),
        pltpu.VMEM((K, HF), jnp.float32),
        pltpu.VMEM((K, HF), jnp.float32),
        pltpu.VMEM((K, HF), jnp.float32),
        pltpu.VMEM_SHARED((N, HF), jnp.float32),
        pltpu.SemaphoreType.DMA,
        pltpu.SemaphoreType.DMA,
        pltpu.SemaphoreType.DMA,
        pltpu.SemaphoreType.DMA,
        pltpu.SemaphoreType.DMA,
        pltpu.SemaphoreType.DMA,
        pltpu.SemaphoreType.DMA,
        pltpu.SemaphoreType.DMA,
    ],
    compiler_params=pltpu.CompilerParams(needs_layout_passes=False,
                                         use_tc_tiling_on_sc=False),
)()


RB = 1000
NB = N // RB


def _mm_body(x_ref, w_ref, o_ref):
    o_ref[...] = jnp.dot(x_ref[...], w_ref[...],
                         preferred_element_type=jnp.float32)


_mm = pl.pallas_call(
    _mm_body,
    grid=(NB,),
    in_specs=[pl.BlockSpec((RB, F), lambda i: (i, 0)),
              pl.BlockSpec((F, F), lambda i: (0, 0))],
    out_specs=pl.BlockSpec((RB, F), lambda i: (i, 0)),
    out_shape=jax.ShapeDtypeStruct((N, F), jnp.float32),
)


def _norm_body(d0_ref, d1_ref, xw_ref, y_ref, dinv_ref):
    deg = d0_ref[0, 0, :] + d1_ref[0, 0, :] + 1.0
    dinv = lax.rsqrt(deg)
    y = xw_ref[...] * dinv[:, None]
    y_ref[0] = y[:, :HF]
    y_ref[1] = y[:, HF:]
    dinv_ref[0, 0, :] = dinv


_norm = pl.pallas_call(
    _norm_body,
    grid=(NB,),
    in_specs=[pl.BlockSpec((1, 1, RB), lambda i: (i, 0, 0)),
              pl.BlockSpec((1, 1, RB), lambda i: (i, 0, 0)),
              pl.BlockSpec((RB, F), lambda i: (i, 0))],
    out_specs=[pl.BlockSpec((NC, RB, HF), lambda i: (0, i, 0)),
               pl.BlockSpec((1, 1, RB), lambda i: (i, 0, 0))],
    out_shape=[jax.ShapeDtypeStruct((NC, N, HF), jnp.float32),
               jax.ShapeDtypeStruct((NB, 1, RB), jnp.float32)],
)


def _mid_body(acc_ref, xw_ref, dinv_ref, b1_ref, g1_ref, be1_ref, w2_ref,
              xw2_ref, y2_ref):
    dinv = dinv_ref[0, 0, :]
    aggs = jnp.concatenate([acc_ref[0], acc_ref[1]], axis=1)
    agg = (aggs * dinv[:, None]
           + xw_ref[...] * (dinv * dinv)[:, None] + b1_ref[...])
    h = jnp.maximum(agg * KBN * g1_ref[...] + be1_ref[...], 0.0)
    xw2 = jnp.dot(h, w2_ref[...], preferred_element_type=jnp.float32)
    xw2_ref[...] = xw2
    y2 = xw2 * dinv[:, None]
    y2_ref[0] = y2[:, :HF]
    y2_ref[1] = y2[:, HF:]


_mid = pl.pallas_call(
    _mid_body,
    grid=(NB,),
    in_specs=[pl.BlockSpec((NC, RB, HF), lambda i: (0, i, 0)),
              pl.BlockSpec((RB, F), lambda i: (i, 0)),
              pl.BlockSpec((1, 1, RB), lambda i: (i, 0, 0)),
              pl.BlockSpec((1, F), lambda i: (0, 0)),
              pl.BlockSpec((1, F), lambda i: (0, 0)),
              pl.BlockSpec((1, F), lambda i: (0, 0)),
              pl.BlockSpec((F, F), lambda i: (0, 0))],
    out_specs=[pl.BlockSpec((RB, F), lambda i: (i, 0)),
               pl.BlockSpec((NC, RB, HF), lambda i: (0, i, 0))],
    out_shape=[jax.ShapeDtypeStruct((N, F), jnp.float32),
               jax.ShapeDtypeStruct((NC, N, HF), jnp.float32)],
)


def _head_body(acc_ref, xw_ref, dinv_ref, b2_ref, g2_ref, be2_ref,
               batch_ref, wc1_ref, bc1_ref, wc2_ref, bc2_ref,
               out_ref, pooled_ref, cnt_ref):
    i = pl.program_id(0)

    @pl.when(i == 0)
    def _():
        pooled_ref[...] = jnp.zeros((G, F), jnp.float32)
        cnt_ref[...] = jnp.zeros((G, F), jnp.float32)

    dinv = dinv_ref[0, 0, :]
    aggs = jnp.concatenate([acc_ref[0], acc_ref[1]], axis=1)
    agg = (aggs * dinv[:, None]
           + xw_ref[...] * (dinv * dinv)[:, None] + b2_ref[...])
    h = jnp.maximum(agg * KBN * g2_ref[...] + be2_ref[...], 0.0)
    b = batch_ref[0, 0, :]
    oneh = (b[:, None] == lax.broadcasted_iota(jnp.int32, (RB, G), 1)
            ).astype(jnp.float32)
    dn = (((0,), (0,)), ((), ()))
    pooled_ref[...] += lax.dot_general(oneh, h, dn,
                                       preferred_element_type=jnp.float32)
    cnt_ref[...] += lax.dot_general(oneh, jnp.ones((RB, F), jnp.float32), dn,
                                    preferred_element_type=jnp.float32)

    @pl.when(i == NB - 1)
    def _():
        pooled = pooled_ref[...] / jnp.maximum(cnt_ref[...], 1.0)
        z = jnp.maximum(
            jnp.dot(pooled, wc1_ref[...], preferred_element_type=jnp.float32)
            + bc1_ref[...], 0.0)
        out_ref[...] = jnp.dot(z, wc2_ref[...],
                               preferred_element_type=jnp.float32) + bc2_ref[...]


_head = pl.pallas_call(
    _head_body,
    grid=(NB,),
    in_specs=[pl.BlockSpec((NC, RB, HF), lambda i: (0, i, 0)),
              pl.BlockSpec((RB, F), lambda i: (i, 0)),
              pl.BlockSpec((1, 1, RB), lambda i: (i, 0, 0)),
              pl.BlockSpec((1, F), lambda i: (0, 0)),
              pl.BlockSpec((1, F), lambda i: (0, 0)),
              pl.BlockSpec((1, F), lambda i: (0, 0)),
              pl.BlockSpec((1, 1, RB), lambda i: (i, 0, 0)),
              pl.BlockSpec((F, F), lambda i: (0, 0)),
              pl.BlockSpec((1, F), lambda i: (0, 0)),
              pl.BlockSpec((F, F), lambda i: (0, 0)),
              pl.BlockSpec((1, F), lambda i: (0, 0))],
    out_specs=pl.BlockSpec((G, F), lambda i: (0, 0)),
    out_shape=jax.ShapeDtypeStruct((G, F), jnp.float32),
    scratch_shapes=[pltpu.VMEM((G, F), jnp.float32),
                    pltpu.VMEM((G, F), jnp.float32)],
)


def kernel(x, edge_index, raw_x, batch, W1, b1, g1, be1, W2, b2, g2, be2,
           Wc1, bc1, Wc2, bc2):
    f32 = jnp.float32
    u = edge_index[0].astype(jnp.int32).reshape(NW, EW_PER)
    v = edge_index[1].astype(jnp.int32).reshape(NW, EW_PER)
    pad = (jnp.arange(PER - EW_PER, dtype=jnp.int32)[None, :] * 89
           + jnp.arange(NW, dtype=jnp.int32)[:, None] * 313) % N
    u3 = jnp.concatenate([u, pad], axis=1)
    v3 = jnp.concatenate([v, pad], axis=1)
    uv3 = (u3 | (v3 << 16)).reshape(NW, CH, K)
    rawx_flat = raw_x.astype(f32).reshape(N * 4)

    xw1 = _mm(x.astype(f32), W1)
    ew3, deg2 = _edge_kernel(uv3, rawx_flat)

    y1, dinv3 = _norm(deg2[0, 0, :N].reshape(NB, 1, RB),
                      deg2[1, 0, :N].reshape(NB, 1, RB), xw1)

    acc1 = _conv_kernel(y1.reshape(NC * N, HF), uv3, ew3)

    r1 = b1.reshape(1, F)
    xw2, y2 = _mid(acc1, xw1, dinv3, r1, g1.reshape(1, F), be1.reshape(1, F),
                   W2)

    acc2 = _conv_kernel(y2.reshape(NC * N, HF), uv3, ew3)

    batch3 = batch.astype(jnp.int32).reshape(NB, 1, RB)
    wc2p = jnp.zeros((F, F), f32).at[:, :C_OUT].set(Wc2)
    bc2p = jnp.zeros((1, F), f32).at[0, :C_OUT].set(bc2)
    out = _head(acc2, xw2, dinv3, b2.reshape(1, F), g2.reshape(1, F),
                be2.reshape(1, F), batch3, Wc1, bc1.reshape(1, F), wc2p, bc2p)
    return out[:, :C_OUT]


# speedup vs baseline: 1.0633x; 1.0101x over previous
import functools
import jax
import jax.numpy as jnp
from jax import lax
from jax.experimental import pallas as pl
from jax.experimental.pallas import tpu as pltpu
from jax.experimental.pallas import tpu_sc as plsc

N = 10000
E = 320000
F = 128
C_OUT = 10
G = 16
SIGMA_SQ = 0.25
KBN = 1.0 / (1.0 + 1e-5) ** 0.5

NC = 2
NS = 16
NW = NC * NS
EW_PER = E // NW
K = 128
CH = 80
PER = CH * K
FULL = EW_PER // K
TAIL = EW_PER - FULL * K

DEG_STRIPE = 640
N_PAD = NS * DEG_STRIPE
ROW_STRIPE = 624
ROW_LAST = N - 15 * ROW_STRIPE
HF = F // 2
CH2 = 2 * CH
T4 = CH2 // 4

_sc_mesh = plsc.VectorSubcoreMesh(core_axis_name="c", subcore_axis_name="s")


def _edge_body(uv3, rawx, ew3, deg2, uv2d, v2d, ew2d, rawx_v, zbuf, deg_acc,
               sem):
    cid = lax.axis_index("c")
    sid = lax.axis_index("s")
    wid = sid * NC + cid

    pltpu.sync_copy(uv3.at[wid], uv2d)
    pltpu.sync_copy(rawx, rawx_v)

    zero = jnp.zeros((16,), jnp.float32)
    for i in range(40):
        zbuf[pl.ds(i * 16, 16)] = zero
    pltpu.sync_copy(zbuf.at[pl.ds(0, DEG_STRIPE)],
                    deg_acc.at[pl.ds(sid * DEG_STRIPE, DEG_STRIPE)])
    plsc.subcore_barrier()

    def chunk(j, _):
        for c in range(K // 16):
            uvv = uv2d[j, pl.ds(c * 16, 16)]
            uvec = uvv & 0xFFFF
            vvec = lax.shift_right_logical(uvv, 16)
            v2d[j, pl.ds(c * 16, 16)] = vvec
            ub = uvec * 4
            vb = vvec * 4
            d2 = jnp.zeros((16,), jnp.float32)
            for k in range(4):
                au = plsc.load_gather(rawx_v, [ub + k])
                av = plsc.load_gather(rawx_v, [vb + k])
                d = au - av
                d2 = d2 + d * d
            ew2d[j, pl.ds(c * 16, 16)] = jnp.exp(d2 * (-0.5 / SIGMA_SQ))
        return 0

    lax.fori_loop(0, CH, chunk, 0)

    for t in range(TAIL // 16, K // 16):
        ew2d[FULL, pl.ds(t * 16, 16)] = zero
    for j in range(FULL + 1, CH):
        for t in range(K // 16):
            ew2d[j, pl.ds(t * 16, 16)] = zero

    def dbatch(t, _):
        for b in range(16):
            pltpu.async_copy(ew2d.at[16 * t + b],
                             deg_acc.at[v2d.at[16 * t + b]], sem, add=True)
        for b in range(16):
            pltpu.make_async_copy(ew2d.at[16 * t + b],
                                  deg_acc.at[v2d.at[16 * t + b]], sem).wait()
        return 0

    lax.fori_loop(0, CH // 16, dbatch, 0)

    pltpu.sync_copy(ew2d, ew3.at[wid])
    plsc.subcore_barrier()

    pltpu.sync_copy(deg_acc.at[pl.ds(sid * DEG_STRIPE, DEG_STRIPE)],
                    deg2.at[cid, 0, pl.ds(sid * DEG_STRIPE, DEG_STRIPE)])


_edge_kernel = functools.partial(
    pl.kernel,
    _edge_body,
    out_type=[jax.ShapeDtypeStruct((NW, CH, K), jnp.float32),
              jax.ShapeDtypeStruct((NC, 1, N_PAD), jnp.float32)],
    mesh=_sc_mesh,
    scratch_types=[
        pltpu.VMEM((CH, K), jnp.int32),
        pltpu.VMEM((CH, K), jnp.int32),
        pltpu.VMEM((CH, K), jnp.float32),
        pltpu.VMEM((N * 4,), jnp.float32),
        pltpu.VMEM((DEG_STRIPE,), jnp.float32),
        pltpu.VMEM_SHARED((N_PAD,), jnp.float32),
        pltpu.SemaphoreType.DMA,
    ],
    compiler_params=pltpu.CompilerParams(needs_layout_passes=False),
)()


def _conv_body(y2, uv3, ew3, out2, uv2d, ew2d, uring, vring, rows0, rows1,
               rows2, rows3, acc, gsem0, gsem1, gsem2, gsem3, ssem0, ssem1,
               ssem2, ssem3):
    cid = lax.axis_index("c")
    sid = lax.axis_index("s")

    pltpu.sync_copy(uv3.at[2 * sid], uv2d.at[pl.ds(0, CH)])
    pltpu.sync_copy(uv3.at[2 * sid + 1], uv2d.at[pl.ds(CH, CH)])
    pltpu.sync_copy(ew3.at[2 * sid], ew2d.at[pl.ds(0, CH)])
    pltpu.sync_copy(ew3.at[2 * sid + 1], ew2d.at[pl.ds(CH, CH)])

    off = cid * N

    def prep(j, slot):
        for g in range(K // 16):
            uvv = uv2d[j, pl.ds(g * 16, 16)]
            uring[slot, pl.ds(g * 16, 16)] = (uvv & 0xFFFF) + off
            vring[slot, pl.ds(g * 16, 16)] = lax.shift_right_logical(uvv, 16)

    zero = jnp.zeros((16,), jnp.float32)

    def zrow(r, _):
        for c in range(HF // 16):
            rows0[r, pl.ds(c * 16, 16)] = zero
        return 0

    lax.fori_loop(0, K, zrow, 0)

    base = sid * ROW_STRIPE

    @pl.when(sid < 15)
    def _():
        for t in range(4):
            pltpu.sync_copy(rows0, acc.at[pl.ds(base + t * K, K)])
        pltpu.sync_copy(rows0.at[pl.ds(0, ROW_STRIPE - 4 * K)],
                        acc.at[pl.ds(base + 4 * K, ROW_STRIPE - 4 * K)])

    @pl.when(sid == 15)
    def _():
        for t in range(5):
            pltpu.sync_copy(rows0, acc.at[pl.ds(base + t * K, K)])
    plsc.subcore_barrier()

    bufs = ((rows0, gsem0, ssem0), (rows1, gsem1, ssem1),
            (rows2, gsem2, ssem2), (rows3, gsem3, ssem3))

    def scale(rows, j):
        @plsc.parallel_loop(0, K // 16, unroll=2)
        def sgroup(g):
            ewv = ew2d[j, pl.ds(g * 16, 16)]
            for r16 in range(16):
                sc = ewv[r16]
                r = g * 16 + r16
                for c in range(HF // 16):
                    rows[r, pl.ds(c * 16, 16)] = (
                        rows[r, pl.ds(c * 16, 16)] * sc)

    prep(0, 0)
    prep(1, 1)
    pltpu.async_copy(y2.at[uring.at[0]], rows0, gsem0)
    pltpu.async_copy(y2.at[uring.at[1]], rows1, gsem1)

    def quad(t, _):
        for b in range(4):
            rows, gsem, ssem = bufs[b]
            ns = (b + 2) % 4
            nrows, ngsem, nssem = bufs[ns]
            j = 4 * t + b
            pltpu.make_async_copy(y2.at[uring.at[b]], rows, gsem).wait()
            if b < 2:
                @pl.when(t > 0)
                def _():
                    pltpu.make_async_copy(
                        nrows, acc.at[vring.at[ns]], nssem).wait()
                prep(j + 2, ns)
                pltpu.async_copy(y2.at[uring.at[ns]], nrows, ngsem)
            else:
                @pl.when(t < T4 - 1)
                def _():
                    pltpu.make_async_copy(
                        nrows, acc.at[vring.at[ns]], nssem).wait()
                    prep(j + 2, ns)
                    pltpu.async_copy(y2.at[uring.at[ns]], nrows, ngsem)
            scale(rows, j)
            pltpu.async_copy(rows, acc.at[vring.at[b]], ssem, add=True)
        return 0

    lax.fori_loop(0, T4, quad, 0)
    for b in range(4):
        rows, gsem, ssem = bufs[b]
        pltpu.make_async_copy(rows, acc.at[vring.at[b]], ssem).wait()
    plsc.subcore_barrier()

    @pl.when(sid < 15)
    def _():
        pltpu.sync_copy(acc.at[pl.ds(base, ROW_STRIPE)],
                        out2.at[cid, pl.ds(base, ROW_STRIPE)])

    @pl.when(sid == 15)
    def _():
        pltpu.sync_copy(acc.at[pl.ds(base, ROW_LAST)],
                        out2.at[cid, pl.ds(base, ROW_LAST)])


_conv_kernel = functools.partial(
    pl.kernel,
    _conv_body,
    out_type=jax.ShapeDtypeStruct((NC, N, HF), jnp.float32),
    mesh=_sc_mesh,
    scratch_types=[
        pltpu.VMEM((CH2, K), jnp.int32),
        pltpu.VMEM((CH2, K), jnp.float32),
        pltpu.VMEM((4, K), jnp.int32),
        pltpu.VMEM((4, K), jnp.int32),
        pltpu.VMEM((K, HF), jnp.float32),
        pltpu.VMEM((K, HF), jnp.float32),
        pltpu.VMEM((K, HF), jnp.float32),
        pltpu.VMEM((K, HF), jnp.float32),
        pltpu.VMEM_SHARED((N, HF), jnp.float32),
        pltpu.SemaphoreType.DMA,
        pltpu.SemaphoreType.DMA,
        pltpu.SemaphoreType.DMA,
        pltpu.SemaphoreType.DMA,
        pltpu.SemaphoreType.DMA,
        pltpu.SemaphoreType.DMA,
        pltpu.SemaphoreType.DMA,
        pltpu.SemaphoreType.DMA,
    ],
    compiler_params=pltpu.CompilerParams(needs_layout_passes=False,
                                         use_tc_tiling_on_sc=False),
)()


RB = 1000
NB = N // RB


def _mm_body(x_ref, w_ref, o_ref):
    o_ref[...] = jnp.dot(x_ref[...], w_ref[...],
                         preferred_element_type=jnp.float32)


_mm = pl.pallas_call(
    _mm_body,
    grid=(NB,),
    in_specs=[pl.BlockSpec((RB, F), lambda i: (i, 0)),
              pl.BlockSpec((F, F), lambda i: (0, 0))],
    out_specs=pl.BlockSpec((RB, F), lambda i: (i, 0)),
    out_shape=jax.ShapeDtypeStruct((N, F), jnp.float32),
)


def _norm_body(d0_ref, d1_ref, xw_ref, y_ref, dinv_ref):
    deg = d0_ref[0, 0, :] + d1_ref[0, 0, :] + 1.0
    dinv = lax.rsqrt(deg)
    y = xw_ref[...] * dinv[:, None]
    y_ref[0] = y[:, :HF]
    y_ref[1] = y[:, HF:]
    dinv_ref[0, 0, :] = dinv


_norm = pl.pallas_call(
    _norm_body,
    grid=(NB,),
    in_specs=[pl.BlockSpec((1, 1, RB), lambda i: (i, 0, 0)),
              pl.BlockSpec((1, 1, RB), lambda i: (i, 0, 0)),
              pl.BlockSpec((RB, F), lambda i: (i, 0))],
    out_specs=[pl.BlockSpec((NC, RB, HF), lambda i: (0, i, 0)),
               pl.BlockSpec((1, 1, RB), lambda i: (i, 0, 0))],
    out_shape=[jax.ShapeDtypeStruct((NC, N, HF), jnp.float32),
               jax.ShapeDtypeStruct((NB, 1, RB), jnp.float32)],
)


def _mid_body(acc_ref, xw_ref, dinv_ref, b1_ref, g1_ref, be1_ref, w2_ref,
              xw2_ref, y2_ref):
    dinv = dinv_ref[0, 0, :]
    aggs = jnp.concatenate([acc_ref[0], acc_ref[1]], axis=1)
    agg = (aggs * dinv[:, None]
           + xw_ref[...] * (dinv * dinv)[:, None] + b1_ref[...])
    h = jnp.maximum(agg * KBN * g1_ref[...] + be1_ref[...], 0.0)
    xw2 = jnp.dot(h, w2_ref[...], preferred_element_type=jnp.float32)
    xw2_ref[...] = xw2
    y2 = xw2 * dinv[:, None]
    y2_ref[0] = y2[:, :HF]
    y2_ref[1] = y2[:, HF:]


_mid = pl.pallas_call(
    _mid_body,
    grid=(NB,),
    in_specs=[pl.BlockSpec((NC, RB, HF), lambda i: (0, i, 0)),
              pl.BlockSpec((RB, F), lambda i: (i, 0)),
              pl.BlockSpec((1, 1, RB), lambda i: (i, 0, 0)),
              pl.BlockSpec((1, F), lambda i: (0, 0)),
              pl.BlockSpec((1, F), lambda i: (0, 0)),
              pl.BlockSpec((1, F), lambda i: (0, 0)),
              pl.BlockSpec((F, F), lambda i: (0, 0))],
    out_specs=[pl.BlockSpec((RB, F), lambda i: (i, 0)),
               pl.BlockSpec((NC, RB, HF), lambda i: (0, i, 0))],
    out_shape=[jax.ShapeDtypeStruct((N, F), jnp.float32),
               jax.ShapeDtypeStruct((NC, N, HF), jnp.float32)],
)


def _head_body(acc_ref, xw_ref, dinv_ref, b2_ref, g2_ref, be2_ref,
               batch_ref, wc1_ref, bc1_ref, wc2_ref, bc2_ref,
               out_ref, pooled_ref, cnt_ref):
    i = pl.program_id(0)

    @pl.when(i == 0)
    def _():
        pooled_ref[...] = jnp.zeros((G, F), jnp.float32)
        cnt_ref[...] = jnp.zeros((G, F), jnp.float32)

    dinv = dinv_ref[0, 0, :]
    aggs = jnp.concatenate([acc_ref[0], acc_ref[1]], axis=1)
    agg = (aggs * dinv[:, None]
           + xw_ref[...] * (dinv * dinv)[:, None] + b2_ref[...])
    h = jnp.maximum(agg * KBN * g2_ref[...] + be2_ref[...], 0.0)
    b = batch_ref[0, 0, :]
    oneh = (b[:, None] == lax.broadcasted_iota(jnp.int32, (RB, G), 1)
            ).astype(jnp.float32)
    dn = (((0,), (0,)), ((), ()))
    pooled_ref[...] += lax.dot_general(oneh, h, dn,
                                       preferred_element_type=jnp.float32)
    cnt_ref[...] += lax.dot_general(oneh, jnp.ones((RB, F), jnp.float32), dn,
                                    preferred_element_type=jnp.float32)

    @pl.when(i == NB - 1)
    def _():
        pooled = pooled_ref[...] / jnp.maximum(cnt_ref[...], 1.0)
        z = jnp.maximum(
            jnp.dot(pooled, wc1_ref[...], preferred_element_type=jnp.float32)
            + bc1_ref[...], 0.0)
        out_ref[...] = jnp.dot(z, wc2_ref[...],
                               preferred_element_type=jnp.float32) + bc2_ref[...]


_head = pl.pallas_call(
    _head_body,
    grid=(NB,),
    in_specs=[pl.BlockSpec((NC, RB, HF), lambda i: (0, i, 0)),
              pl.BlockSpec((RB, F), lambda i: (i, 0)),
              pl.BlockSpec((1, 1, RB), lambda i: (i, 0, 0)),
              pl.BlockSpec((1, F), lambda i: (0, 0)),
              pl.BlockSpec((1, F), lambda i: (0, 0)),
              pl.BlockSpec((1, F), lambda i: (0, 0)),
              pl.BlockSpec((1, 1, RB), lambda i: (i, 0, 0)),
              pl.BlockSpec((F, F), lambda i: (0, 0)),
              pl.BlockSpec((1, F), lambda i: (0, 0)),
              pl.BlockSpec((F, F), lambda i: (0, 0)),
              pl.BlockSpec((1, F), lambda i: (0, 0))],
    out_specs=pl.BlockSpec((G, F), lambda i: (0, 0)),
    out_shape=jax.ShapeDtypeStruct((G, F), jnp.float32),
    scratch_shapes=[pltpu.VMEM((G, F), jnp.float32),
                    pltpu.VMEM((G, F), jnp.float32)],
)


def kernel(x, edge_index, raw_x, batch, W1, b1, g1, be1, W2, b2, g2, be2,
           Wc1, bc1, Wc2, bc2):
    f32 = jnp.float32
    u = edge_index[0].astype(jnp.int32).reshape(NW, EW_PER)
    v = edge_index[1].astype(jnp.int32).reshape(NW, EW_PER)
    pad = (jnp.arange(PER - EW_PER, dtype=jnp.int32)[None, :] * 89
           + jnp.arange(NW, dtype=jnp.int32)[:, None] * 313) % N
    u3 = jnp.concatenate([u, pad], axis=1)
    v3 = jnp.concatenate([v, pad], axis=1)
    uv3 = (u3 | (v3 << 16)).reshape(NW, CH, K)
    rawx_flat = raw_x.astype(f32).reshape(N * 4)

    xw1 = _mm(x.astype(f32), W1)
    ew3, deg2 = _edge_kernel(uv3, rawx_flat)

    y1, dinv3 = _norm(deg2[0, 0, :N].reshape(NB, 1, RB),
                      deg2[1, 0, :N].reshape(NB, 1, RB), xw1)

    acc1 = _conv_kernel(y1.reshape(NC * N, HF), uv3, ew3)

    r1 = b1.reshape(1, F)
    xw2, y2 = _mid(acc1, xw1, dinv3, r1, g1.reshape(1, F), be1.reshape(1, F),
                   W2)

    acc2 = _conv_kernel(y2.reshape(NC * N, HF), uv3, ew3)

    batch3 = batch.astype(jnp.int32).reshape(NB, 1, RB)
    wc2p = jnp.zeros((F, F), f32).at[:, :C_OUT].set(Wc2)
    bc2p = jnp.zeros((1, F), f32).at[0, :C_OUT].set(bc2)
    out = _head(acc2, xw2, dinv3, b2.reshape(1, F), g2.reshape(1, F),
                be2.reshape(1, F), batch3, Wc1, bc1.reshape(1, F), wc2p, bc2p)
    return out[:, :C_OUT]
